# Initial kernel scaffold; baseline (speedup 1.0000x reference)
import jax
import jax.numpy as jnp
from jax.experimental import pallas as pl


def kernel(x, edge_index, edge_attr, u, batch, W_in, b_in, W_hid, b_hid, W_out, gamma, beta):
    raise NotImplementedError("write your pallas kernel here")


# speedup vs baseline: 5.4873x; 5.4873x over previous
import functools

import jax
import jax.numpy as jnp
from jax import lax
from jax.experimental import pallas as pl
from jax.experimental.pallas import tpu as pltpu
from jax.experimental.pallas import tpu_sc as plsc

N = 10000
E = 160000
B = 64
DF = 256
DE = 16
GIN = 64
HID = 512
OUT = 256
NHID = 3

NC = 2
NS = 16
NW = NC * NS
CH = 128
NCHUNK = E // CH
TILE_C = 40
NCPAD = NW * TILE_C
ALOAD = 1280
CPAD = 640
NROW = 10240
RPT = NROW // NS


def _sc_edge_scatter(dst2d, attr):
    mesh = plsc.VectorSubcoreMesh(core_axis_name="c", subcore_axis_name="s")

    @functools.partial(
        pl.kernel,
        out_type=(
            jax.ShapeDtypeStruct((NC, NROW, DE), jnp.float32),
            jax.ShapeDtypeStruct((NC, NROW, DE), jnp.float32),
        ),
        mesh=mesh,
        compiler_params=pltpu.CompilerParams(use_tc_tiling_on_sc=False),
        scratch_types=[
            pltpu.VMEM((TILE_C, CH), jnp.int32),
            pltpu.VMEM((TILE_C * CH, DE), jnp.float32),
            pltpu.VMEM((RPT, DE), jnp.float32),
            pltpu.VMEM((CH, DE), jnp.float32),
            pltpu.VMEM_SHARED((NROW, DE), jnp.float32),
            pltpu.VMEM_SHARED((NROW, DE), jnp.float32),
        ],
    )
    def k(dst_hbm, attr_hbm, val_out, cnt_out,
          idx_all, attr_all, zrow, ones_rows,
          val_sh, cnt_sh):
        c = lax.axis_index("c")
        s = lax.axis_index("s")
        wid = c * NS + s

        zeros16 = jnp.zeros((16,), jnp.float32)
        lane = lax.iota(jnp.int32, 16)
        onehot = jnp.where(lane == 0, 1.0, 0.0).astype(jnp.float32)

        def _zero_zrow(i, carry):
            zrow[i, :] = zeros16
            return carry
        lax.fori_loop(0, RPT, _zero_zrow, 0)

        def _fill_ones(i, carry):
            ones_rows[i, :] = onehot
            return carry
        lax.fori_loop(0, CH, _fill_ones, 0)

        pltpu.sync_copy(zrow, val_sh.at[pl.ds(s * RPT, RPT)])
        pltpu.sync_copy(zrow, cnt_sh.at[pl.ds(s * RPT, RPT)])

        plsc.subcore_barrier()

        pltpu.sync_copy(dst_hbm.at[pl.ds(wid * TILE_C, TILE_C)], idx_all)
        for kk in range(TILE_C * CH // ALOAD):
            row0 = wid * TILE_C * CH + kk * ALOAD

            @pl.when(row0 + ALOAD <= E)
            def _():
                pltpu.sync_copy(attr_hbm.at[pl.ds(row0, ALOAD)],
                                attr_all.at[pl.ds(kk * ALOAD, ALOAD)])

        def _chunk(ci, carry):
            @pl.when(wid * TILE_C + ci < NCHUNK)
            def _():
                pltpu.sync_copy(attr_all.at[pl.ds(ci * CH, CH)],
                                val_sh.at[idx_all.at[ci]], add=True)
                pltpu.sync_copy(ones_rows,
                                cnt_sh.at[idx_all.at[ci]], add=True)
            return carry
        lax.fori_loop(0, TILE_C, _chunk, 0)

        plsc.subcore_barrier()

        pltpu.sync_copy(val_sh.at[pl.ds(s * RPT, RPT)],
                        val_out.at[c, pl.ds(s * RPT, RPT)])
        pltpu.sync_copy(cnt_sh.at[pl.ds(s * RPT, RPT)],
                        cnt_out.at[c, pl.ds(s * RPT, RPT)])

    return k(dst2d, attr)


NBLK = 10
CHN = N // NBLK


def _elu(v):
    return jnp.where(v > 0, v, jnp.exp(v) - 1.0)


def _tc_body(x_ref, b_ref, u_ref, svc_ref, wu_ref, wn_ref, we_ref, bin_ref,
             wh_ref, bh_ref, wo_ref, g_ref, beta_ref, out_ref,
             accx, acce, accc):
    i = pl.program_id(0)

    @pl.when(i == 0)
    def _():
        accx[...] = jnp.zeros_like(accx)
        acce[...] = jnp.zeros_like(acce)
        accc[...] = jnp.zeros_like(accc)

    bblk = b_ref[0]
    iota = lax.broadcasted_iota(jnp.int32, (B, CHN), 0)
    mask = (bblk == iota).astype(jnp.float32)

    accx[...] += jnp.dot(mask, x_ref[...], preferred_element_type=jnp.float32)

    sv = svc_ref[...]
    ev = sv[:, 0:DE] + sv[:, DE:2 * DE]
    ec = sv[:, 2 * DE:2 * DE + 1] + sv[:, 2 * DE + 1:2 * DE + 2]
    em = ev / jnp.maximum(ec, 1.0)
    acce[...] += jnp.dot(mask, em, preferred_element_type=jnp.float32)
    accc[...] += jnp.sum(mask, axis=1, keepdims=True)

    @pl.when(i == NBLK - 1)
    def _():
        cnt = jnp.maximum(accc[...], 1.0)
        node_mean = accx[...] / cnt
        edge_mean = acce[...] / cnt
        h = _elu(jnp.dot(u_ref[...], wu_ref[...],
                         preferred_element_type=jnp.float32)
                 + jnp.dot(node_mean, wn_ref[...],
                           preferred_element_type=jnp.float32)
                 + jnp.dot(edge_mean, we_ref[...],
                           preferred_element_type=jnp.float32)
                 + bin_ref[...])
        skip = h
        for l in range(NHID):
            h = _elu(jnp.dot(h, wh_ref[l],
                             preferred_element_type=jnp.float32)
                     + bh_ref[l])
        h = h + skip
        z = jnp.dot(h, wo_ref[...], preferred_element_type=jnp.float32)
        mu = jnp.mean(z, axis=-1, keepdims=True)
        var = jnp.mean((z - mu) ** 2, axis=-1, keepdims=True)
        z = (z - mu) / jnp.sqrt(var + 1e-5) * g_ref[...] + beta_ref[...]
        out_ref[...] = _elu(z)


def _tc_main(x, batch3d, u, svc, wu, wn, we, bin2, wh, bh3, wo, g2, beta2):
    const = lambda shape: pl.BlockSpec(shape, lambda i: tuple(0 for _ in shape))
    return pl.pallas_call(
        _tc_body,
        grid=(NBLK,),
        in_specs=[
            pl.BlockSpec((CHN, DF), lambda i: (i, 0)),
            pl.BlockSpec((1, 1, CHN), lambda i: (i, 0, 0)),
            const((B, GIN)),
            pl.BlockSpec((CHN, 2 * DE + 2), lambda i: (i, 0)),
            const((GIN, HID)),
            const((DF, HID)),
            const((DE, HID)),
            const((1, HID)),
            const((NHID, HID, HID)),
            const((NHID, 1, HID)),
            const((HID, OUT)),
            const((1, OUT)),
            const((1, OUT)),
        ],
        out_specs=pl.BlockSpec((B, OUT), lambda i: (0, 0)),
        out_shape=jax.ShapeDtypeStruct((B, OUT), jnp.float32),
        scratch_shapes=[
            pltpu.VMEM((B, DF), jnp.float32),
            pltpu.VMEM((B, DE), jnp.float32),
            pltpu.VMEM((B, 1), jnp.float32),
        ],
    )(x, batch3d, u, svc, wu, wn, we, bin2, wh, bh3, wo, g2, beta2)


def kernel(x, edge_index, edge_attr, u, batch, W_in, b_in, W_hid, b_hid,
           W_out, gamma, beta):
    dst2d = jnp.pad(edge_index[1].reshape(NCHUNK, CH),
                    ((0, NCPAD - NCHUNK), (0, 0)))
    val, cnt = _sc_edge_scatter(dst2d, edge_attr)

    svc = jnp.concatenate(
        [val[0, :N], val[1, :N], cnt[0, :N, :1], cnt[1, :N, :1]], axis=1)

    batch3d = batch.reshape(NBLK, 1, CHN)
    wu = W_in[:, :GIN].T
    wn = W_in[:, GIN:GIN + DF].T
    we = W_in[:, GIN + DF:].T
    bin2 = b_in[None, :]
    wh = jnp.transpose(W_hid, (0, 2, 1))
    bh3 = b_hid[:, None, :]
    wo = W_out.T
    g2 = gamma[None, :]
    beta2 = beta[None, :]

    return _tc_main(x, batch3d, u, svc, wu, wn, we, bin2, wh, bh3, wo,
                    g2, beta2)


# speedup vs baseline: 6.1554x; 1.1218x over previous
import functools

import jax
import jax.numpy as jnp
from jax import lax
from jax.experimental import pallas as pl
from jax.experimental.pallas import tpu as pltpu
from jax.experimental.pallas import tpu_sc as plsc

N = 10000
E = 160000
B = 64
DF = 256
DE = 16
GIN = 64
HID = 512
OUT = 256
NHID = 3

NC = 2
NS = 16
NW = NC * NS
CH = 128
NCHUNK = E // CH
TILE_C = 40
NCPAD = NW * TILE_C
ALOAD = 1280
CPAD = 640
NROW = 10240
RPT = NROW // NS


def _sc_edge_scatter(dst2d, attr):
    mesh = plsc.VectorSubcoreMesh(core_axis_name="c", subcore_axis_name="s")

    @functools.partial(
        pl.kernel,
        out_type=(
            jax.ShapeDtypeStruct((NC, NROW, DE), jnp.float32),
            jax.ShapeDtypeStruct((NC, NROW, DE), jnp.float32),
        ),
        mesh=mesh,
        compiler_params=pltpu.CompilerParams(use_tc_tiling_on_sc=False),
        scratch_types=[
            pltpu.VMEM((TILE_C, CH), jnp.int32),
            pltpu.VMEM((TILE_C * CH, DE), jnp.float32),
            pltpu.VMEM((RPT, DE), jnp.float32),
            pltpu.VMEM((CH, DE), jnp.float32),
            pltpu.VMEM_SHARED((NROW, DE), jnp.float32),
            pltpu.VMEM_SHARED((NROW, DE), jnp.float32),
        ],
    )
    def k(dst_hbm, attr_hbm, val_out, cnt_out,
          idx_all, attr_all, zrow, ones_rows,
          val_sh, cnt_sh):
        c = lax.axis_index("c")
        s = lax.axis_index("s")
        wid = c * NS + s

        zeros16 = jnp.zeros((16,), jnp.float32)
        lane = lax.iota(jnp.int32, 16)
        onehot = jnp.where(lane == 0, 1.0, 0.0).astype(jnp.float32)

        def _zero_zrow(i, carry):
            zrow[i, :] = zeros16
            return carry
        lax.fori_loop(0, RPT, _zero_zrow, 0)

        def _fill_ones(i, carry):
            ones_rows[i, :] = onehot
            return carry
        lax.fori_loop(0, CH, _fill_ones, 0)

        pltpu.sync_copy(zrow, val_sh.at[pl.ds(s * RPT, RPT)])
        pltpu.sync_copy(zrow, cnt_sh.at[pl.ds(s * RPT, RPT)])

        plsc.subcore_barrier()

        pltpu.sync_copy(dst_hbm.at[pl.ds(wid * TILE_C, TILE_C)], idx_all)
        for kk in range(TILE_C * CH // ALOAD):
            row0 = wid * TILE_C * CH + kk * ALOAD

            @pl.when(row0 + ALOAD <= E)
            def _():
                pltpu.sync_copy(attr_hbm.at[pl.ds(row0, ALOAD)],
                                attr_all.at[pl.ds(kk * ALOAD, ALOAD)])

        def _chunk(ci, carry):
            @pl.when(wid * TILE_C + ci < NCHUNK)
            def _():
                pltpu.sync_copy(attr_all.at[pl.ds(ci * CH, CH)],
                                val_sh.at[idx_all.at[ci]], add=True)
                pltpu.sync_copy(ones_rows,
                                cnt_sh.at[idx_all.at[ci]], add=True)
            return carry
        lax.fori_loop(0, TILE_C, _chunk, 0)

        plsc.subcore_barrier()

        pltpu.sync_copy(val_sh.at[pl.ds(s * RPT, RPT)],
                        val_out.at[c, pl.ds(s * RPT, RPT)])
        pltpu.sync_copy(cnt_sh.at[pl.ds(s * RPT, RPT)],
                        cnt_out.at[c, pl.ds(s * RPT, RPT)])

    return k(dst2d, attr)


NBLK = 10
CHN = N // NBLK


def _elu(v):
    return jnp.where(v > 0, v, jnp.exp(v) - 1.0)


def _dot_nt(a, w):
    return lax.dot_general(a, w, (((1,), (1,)), ((), ())),
                           preferred_element_type=jnp.float32)


def _tc_body(x_ref, b_ref, u_ref, val_ref, cnt_ref, win_ref, bin_ref,
             wh_ref, bh_ref, wo_ref, g_ref, beta_ref, out_ref,
             accx, acce, accc):
    i = pl.program_id(0)

    @pl.when(i == 0)
    def _():
        accx[...] = jnp.zeros_like(accx)
        acce[...] = jnp.zeros_like(acce)
        accc[...] = jnp.zeros_like(accc)

    bblk = b_ref[0]
    iota = lax.broadcasted_iota(jnp.int32, (B, CHN), 0)
    mask = (bblk == iota).astype(jnp.float32)

    accx[...] += jnp.dot(mask, x_ref[...], preferred_element_type=jnp.float32)

    ev = val_ref[0] + val_ref[1]
    ec = cnt_ref[0, :, 0:1] + cnt_ref[1, :, 0:1]
    em = ev / jnp.maximum(ec, 1.0)
    acce[...] += jnp.dot(mask, em, preferred_element_type=jnp.float32)
    accc[...] += jnp.sum(mask, axis=1, keepdims=True)

    @pl.when(i == NBLK - 1)
    def _():
        cnt = jnp.maximum(accc[...], 1.0)
        node_mean = accx[...] / cnt
        edge_mean = acce[...] / cnt
        feat = jnp.concatenate([u_ref[...], node_mean, edge_mean], axis=1)
        h = _elu(_dot_nt(feat, win_ref[...]) + bin_ref[...])
        skip = h
        for l in range(NHID):
            h = _elu(_dot_nt(h, wh_ref[l]) + bh_ref[l])
        h = h + skip
        z = _dot_nt(h, wo_ref[...])
        mu = jnp.mean(z, axis=-1, keepdims=True)
        var = jnp.mean((z - mu) ** 2, axis=-1, keepdims=True)
        z = (z - mu) / jnp.sqrt(var + 1e-5) * g_ref[...] + beta_ref[...]
        out_ref[...] = _elu(z)


def _tc_main(x, batch3d, u, val, cnt, W_in, bin2, W_hid, bh3, W_out,
             g2, beta2):
    const = lambda shape: pl.BlockSpec(shape, lambda i: tuple(0 for _ in shape))
    return pl.pallas_call(
        _tc_body,
        grid=(NBLK,),
        in_specs=[
            pl.BlockSpec((CHN, DF), lambda i: (i, 0)),
            pl.BlockSpec((1, 1, CHN), lambda i: (i, 0, 0)),
            const((B, GIN)),
            pl.BlockSpec((NC, CHN, DE), lambda i: (0, i, 0)),
            pl.BlockSpec((NC, CHN, DE), lambda i: (0, i, 0)),
            const((HID, GIN + DF + DE)),
            const((1, HID)),
            const((NHID, HID, HID)),
            const((NHID, 1, HID)),
            const((OUT, HID)),
            const((1, OUT)),
            const((1, OUT)),
        ],
        out_specs=pl.BlockSpec((B, OUT), lambda i: (0, 0)),
        out_shape=jax.ShapeDtypeStruct((B, OUT), jnp.float32),
        scratch_shapes=[
            pltpu.VMEM((B, DF), jnp.float32),
            pltpu.VMEM((B, DE), jnp.float32),
            pltpu.VMEM((B, 1), jnp.float32),
        ],
    )(x, batch3d, u, val, cnt, W_in, bin2, W_hid, bh3, W_out, g2, beta2)


def kernel(x, edge_index, edge_attr, u, batch, W_in, b_in, W_hid, b_hid,
           W_out, gamma, beta):
    dst2d = jnp.pad(edge_index[1].reshape(NCHUNK, CH),
                    ((0, NCPAD - NCHUNK), (0, 0)))
    val, cnt = _sc_edge_scatter(dst2d, edge_attr)

    batch3d = batch.reshape(NBLK, 1, CHN)
    return _tc_main(x, batch3d, u, val, cnt, W_in, b_in[None, :], W_hid,
                    b_hid[:, None, :], W_out, gamma[None, :], beta[None, :])


# speedup vs baseline: 7.0970x; 1.1530x over previous
import functools

import jax
import jax.numpy as jnp
from jax import lax
from jax.experimental import pallas as pl
from jax.experimental.pallas import tpu as pltpu
from jax.experimental.pallas import tpu_sc as plsc

N = 10000
E = 160000
B = 64
DF = 256
DE = 16
GIN = 64
HID = 512
OUT = 256
NHID = 3

NC = 2
NS = 16
NW = NC * NS
CH = 125
NCHUNK = E // CH
TILE_C = NCHUNK // NW
EPT = TILE_C * CH
NROW = 10240
RPT = NROW // NS


def _sc_edge_scatter(ei3d, attr):
    mesh = plsc.VectorSubcoreMesh(core_axis_name="c", subcore_axis_name="s")

    @functools.partial(
        pl.kernel,
        out_type=(
            jax.ShapeDtypeStruct((NC, NROW, DE), jnp.float32),
            jax.ShapeDtypeStruct((NC, NROW, DE), jnp.float32),
        ),
        mesh=mesh,
        compiler_params=pltpu.CompilerParams(use_tc_tiling_on_sc=False),
        scratch_types=[
            pltpu.VMEM((TILE_C, CH), jnp.int32),
            pltpu.VMEM((EPT, DE), jnp.float32),
            pltpu.VMEM((RPT, DE), jnp.float32),
            pltpu.VMEM((CH, DE), jnp.float32),
            pltpu.VMEM_SHARED((NROW, DE), jnp.float32),
            pltpu.VMEM_SHARED((NROW, DE), jnp.float32),
        ],
    )
    def k(dst_hbm, attr_hbm, val_out, cnt_out,
          idx_all, attr_all, zrow, ones_rows,
          val_sh, cnt_sh):
        c = lax.axis_index("c")
        s = lax.axis_index("s")
        wid = c * NS + s

        zeros16 = jnp.zeros((16,), jnp.float32)
        lane = lax.iota(jnp.int32, 16)
        onehot = jnp.where(lane == 0, 1.0, 0.0).astype(jnp.float32)

        def _zero_zrow(i, carry):
            zrow[i, :] = zeros16
            return carry
        lax.fori_loop(0, RPT, _zero_zrow, 0)

        def _fill_ones(i, carry):
            ones_rows[i, :] = onehot
            return carry
        lax.fori_loop(0, CH, _fill_ones, 0)

        pltpu.sync_copy(zrow, val_sh.at[pl.ds(s * RPT, RPT)])
        pltpu.sync_copy(zrow, cnt_sh.at[pl.ds(s * RPT, RPT)])

        plsc.subcore_barrier()

        pltpu.sync_copy(dst_hbm.at[1, pl.ds(wid * TILE_C, TILE_C)], idx_all)
        pltpu.sync_copy(attr_hbm.at[pl.ds(wid * EPT, EPT)], attr_all)

        def _chunk(ci, carry):
            pltpu.sync_copy(attr_all.at[pl.ds(ci * CH, CH)],
                            val_sh.at[idx_all.at[ci]], add=True)
            pltpu.sync_copy(ones_rows,
                            cnt_sh.at[idx_all.at[ci]], add=True)
            return carry
        lax.fori_loop(0, TILE_C, _chunk, 0)

        plsc.subcore_barrier()

        pltpu.sync_copy(val_sh.at[pl.ds(s * RPT, RPT)],
                        val_out.at[c, pl.ds(s * RPT, RPT)])
        pltpu.sync_copy(cnt_sh.at[pl.ds(s * RPT, RPT)],
                        cnt_out.at[c, pl.ds(s * RPT, RPT)])

    return k(ei3d, attr)


NBLK = 10
CHN = N // NBLK


def _elu(v):
    return jnp.where(v > 0, v, jnp.exp(v) - 1.0)


def _dot_nt(a, w):
    return lax.dot_general(a, w, (((1,), (1,)), ((), ())),
                           preferred_element_type=jnp.float32)


NRV = NROW * DE // 128
NPK = 128 // DE


def _tc_body(x_ref, b_ref, b8_ref, u_ref, val_ref, cnt_ref, win_ref, bin_ref,
             wh_ref, bh_ref, wo_ref, g_ref, beta_ref, out_ref,
             accx, acce, accc):
    i = pl.program_id(0)

    @pl.when(i == 0)
    def _():
        accx[...] = jnp.zeros_like(accx)
        c128 = cnt_ref[0] + cnt_ref[1]
        kk = lax.broadcasted_iota(jnp.int32, (128, 128), 0)
        ll = lax.broadcasted_iota(jnp.int32, (128, 128), 1)
        sel = (kk == (ll // DE) * DE).astype(jnp.float32)
        cb = jnp.dot(c128, sel, preferred_element_type=jnp.float32)
        em128 = (val_ref[0] + val_ref[1]) / jnp.maximum(cb, 1.0)
        iota8 = lax.broadcasted_iota(jnp.int32, (B, NRV), 0)
        ae = jnp.zeros((B, DE), jnp.float32)
        ac = jnp.zeros((B, 1), jnp.float32)
        for a in range(NPK):
            mask_a = (b8_ref[a:a + 1, :] == iota8).astype(jnp.float32)
            ae = ae + jnp.dot(mask_a, em128[:, a * DE:(a + 1) * DE],
                              preferred_element_type=jnp.float32)
            ac = ac + jnp.sum(mask_a, axis=1, keepdims=True)
        acce[...] = ae
        accc[...] = ac

    bblk = b_ref[0]
    iota = lax.broadcasted_iota(jnp.int32, (B, CHN), 0)
    mask = (bblk == iota).astype(jnp.float32)

    accx[...] += jnp.dot(mask, x_ref[...], preferred_element_type=jnp.float32)

    @pl.when(i == NBLK - 1)
    def _():
        cnt = jnp.maximum(accc[...], 1.0)
        node_mean = accx[...] / cnt
        edge_mean = acce[...] / cnt
        feat = jnp.concatenate([u_ref[...], node_mean, edge_mean], axis=1)
        h = _elu(_dot_nt(feat, win_ref[...]) + bin_ref[...])
        skip = h
        for l in range(NHID):
            h = _elu(_dot_nt(h, wh_ref[l]) + bh_ref[l])
        h = h + skip
        z = _dot_nt(h, wo_ref[...])
        mu = jnp.mean(z, axis=-1, keepdims=True)
        var = jnp.mean((z - mu) ** 2, axis=-1, keepdims=True)
        z = (z - mu) / jnp.sqrt(var + 1e-5) * g_ref[...] + beta_ref[...]
        out_ref[...] = _elu(z)


def _tc_main(x, batch3d, batch8, u, val, cnt, W_in, bin2, W_hid, bh3, W_out,
             g2, beta2):
    const = lambda shape: pl.BlockSpec(shape, lambda i: tuple(0 for _ in shape))
    return pl.pallas_call(
        _tc_body,
        grid=(NBLK,),
        in_specs=[
            pl.BlockSpec((CHN, DF), lambda i: (i, 0)),
            pl.BlockSpec((1, 1, CHN), lambda i: (i, 0, 0)),
            const((NPK, NRV)),
            const((B, GIN)),
            const((NC, NROW * DE // 128, 128)),
            const((NC, NROW * DE // 128, 128)),
            const((HID, GIN + DF + DE)),
            const((1, HID)),
            const((NHID, HID, HID)),
            const((NHID, 1, HID)),
            const((OUT, HID)),
            const((1, OUT)),
            const((1, OUT)),
        ],
        out_specs=pl.BlockSpec((B, OUT), lambda i: (0, 0)),
        out_shape=jax.ShapeDtypeStruct((B, OUT), jnp.float32),
        scratch_shapes=[
            pltpu.VMEM((B, DF), jnp.float32),
            pltpu.VMEM((B, DE), jnp.float32),
            pltpu.VMEM((B, 1), jnp.float32),
        ],
    )(x, batch3d, batch8, u, val, cnt, W_in, bin2, W_hid, bh3, W_out,
      g2, beta2)


def kernel(x, edge_index, edge_attr, u, batch, W_in, b_in, W_hid, b_hid,
           W_out, gamma, beta):
    ei3d = edge_index.reshape(2, NCHUNK, CH)
    val, cnt = _sc_edge_scatter(ei3d, edge_attr)
    val = val.reshape(NC, NROW * DE // 128, 128)
    cnt = cnt.reshape(NC, NROW * DE // 128, 128)

    batch3d = batch.reshape(NBLK, 1, CHN)
    batch8 = jnp.pad(batch, (0, NROW - N), constant_values=B)
    batch8 = batch8.reshape(NROW * DE // 128, 128 // DE).T
    return _tc_main(x, batch3d, batch8, u, val, cnt, W_in, b_in[None, :],
                    W_hid, b_hid[:, None, :], W_out, gamma[None, :],
                    beta[None, :])


# speedup vs baseline: 7.4015x; 1.0429x over previous
import functools

import jax
import jax.numpy as jnp
from jax import lax
from jax.experimental import pallas as pl
from jax.experimental.pallas import tpu as pltpu
from jax.experimental.pallas import tpu_sc as plsc

N = 10000
E = 160000
B = 64
DF = 256
DE = 16
GIN = 64
HID = 512
OUT = 256
NHID = 3

NC = 2
NS = 16
NW = NC * NS
CH = 125
NCHUNK = E // CH
TILE_C = NCHUNK // NW
EPT = TILE_C * CH
NROW = 10240
RPT = NROW // NS


def _sc_edge_scatter(ei3d, attr):
    mesh = plsc.VectorSubcoreMesh(core_axis_name="c", subcore_axis_name="s")

    @functools.partial(
        pl.kernel,
        out_type=(
            jax.ShapeDtypeStruct((NC, NROW, DE), jnp.float32),
            jax.ShapeDtypeStruct((NC, NROW, DE), jnp.float32),
        ),
        mesh=mesh,
        compiler_params=pltpu.CompilerParams(use_tc_tiling_on_sc=False),
        scratch_types=[
            pltpu.VMEM((TILE_C, CH), jnp.int32),
            pltpu.VMEM((EPT, DE), jnp.float32),
            pltpu.VMEM((RPT, DE), jnp.float32),
            pltpu.VMEM((CH, DE), jnp.float32),
            pltpu.VMEM_SHARED((NROW, DE), jnp.float32),
            pltpu.VMEM_SHARED((NROW, DE), jnp.float32),
        ],
    )
    def k(dst_hbm, attr_hbm, val_out, cnt_out,
          idx_all, attr_all, zrow, ones_rows,
          val_sh, cnt_sh):
        c = lax.axis_index("c")
        s = lax.axis_index("s")
        wid = c * NS + s

        zeros16 = jnp.zeros((16,), jnp.float32)
        lane = lax.iota(jnp.int32, 16)
        onehot = jnp.where(lane == 0, 1.0, 0.0).astype(jnp.float32)

        def _zero_zrow(i, carry):
            zrow[i, :] = zeros16
            return carry
        lax.fori_loop(0, RPT, _zero_zrow, 0)

        def _fill_ones(i, carry):
            ones_rows[i, :] = onehot
            return carry
        lax.fori_loop(0, CH, _fill_ones, 0)

        pltpu.sync_copy(zrow, val_sh.at[pl.ds(s * RPT, RPT)])
        pltpu.sync_copy(zrow, cnt_sh.at[pl.ds(s * RPT, RPT)])

        plsc.subcore_barrier()

        pltpu.sync_copy(dst_hbm.at[1, pl.ds(wid * TILE_C, TILE_C)], idx_all)
        pltpu.sync_copy(attr_hbm.at[pl.ds(wid * EPT, EPT)], attr_all)

        def _chunk(ci, carry):
            pltpu.sync_copy(attr_all.at[pl.ds(ci * CH, CH)],
                            val_sh.at[idx_all.at[ci]], add=True)
            pltpu.sync_copy(ones_rows,
                            cnt_sh.at[idx_all.at[ci]], add=True)
            return carry
        lax.fori_loop(0, TILE_C, _chunk, 0)

        plsc.subcore_barrier()

        pltpu.sync_copy(val_sh.at[pl.ds(s * RPT, RPT)],
                        val_out.at[c, pl.ds(s * RPT, RPT)])
        pltpu.sync_copy(cnt_sh.at[pl.ds(s * RPT, RPT)],
                        cnt_out.at[c, pl.ds(s * RPT, RPT)])

    return k(ei3d, attr)


NBLK = 5
CHN = N // NBLK
NRV = NROW * DE // 128
NPK = 128 // DE


def _elu(v):
    return jnp.where(v > 0, v, jnp.exp(v) - 1.0)


def _dot_nt(a, w):
    return lax.dot_general(a, w, (((1,), (1,)), ((), ())),
                           preferred_element_type=jnp.float32)


def _tc1_body(x_ref, b_ref, xs_ref, cs_ref, accx, accc):
    i = pl.program_id(0)

    @pl.when(i == 0)
    def _():
        accx[...] = jnp.zeros_like(accx)
        accc[...] = jnp.zeros_like(accc)

    bblk = b_ref[0]
    iota = lax.broadcasted_iota(jnp.int32, (B, CHN), 0)
    mask = (bblk == iota).astype(jnp.float32)
    accx[...] += jnp.dot(mask, x_ref[...], preferred_element_type=jnp.float32)
    accc[...] += jnp.sum(mask, axis=1, keepdims=True)

    @pl.when(i == NBLK - 1)
    def _():
        xs_ref[...] = accx[...]
        cs_ref[...] = jnp.broadcast_to(accc[...], (B, 128))


def _tc1(x, batch3d):
    return pl.pallas_call(
        _tc1_body,
        grid=(NBLK,),
        in_specs=[
            pl.BlockSpec((CHN, DF), lambda i: (i, 0)),
            pl.BlockSpec((1, 1, CHN), lambda i: (i, 0, 0)),
        ],
        out_specs=[pl.BlockSpec((B, DF), lambda i: (0, 0)),
                   pl.BlockSpec((B, 128), lambda i: (0, 0))],
        out_shape=[jax.ShapeDtypeStruct((B, DF), jnp.float32),
                   jax.ShapeDtypeStruct((B, 128), jnp.float32)],
        scratch_shapes=[
            pltpu.VMEM((B, DF), jnp.float32),
            pltpu.VMEM((B, 1), jnp.float32),
        ],
    )(x, batch3d)


def _tc2_body(b8_ref, u_ref, val_ref, cnt_ref, xs_ref, cs_ref, win_ref,
              bin_ref, wh_ref, bh_ref, wo_ref, g_ref, beta_ref, out_ref):
    c128 = cnt_ref[0] + cnt_ref[1]
    kk = lax.broadcasted_iota(jnp.int32, (128, 128), 0)
    ll = lax.broadcasted_iota(jnp.int32, (128, 128), 1)
    sel = (kk == (ll // DE) * DE).astype(jnp.float32)
    cb = jnp.dot(c128, sel, preferred_element_type=jnp.float32)
    em128 = (val_ref[0] + val_ref[1]) / jnp.maximum(cb, 1.0)
    iota8 = lax.broadcasted_iota(jnp.int32, (B, NRV), 0)
    ae = jnp.zeros((B, DE), jnp.float32)
    for a in range(NPK):
        mask_a = (b8_ref[a:a + 1, :] == iota8).astype(jnp.float32)
        ae = ae + jnp.dot(mask_a, em128[:, a * DE:(a + 1) * DE],
                          preferred_element_type=jnp.float32)

    cnt = jnp.maximum(cs_ref[:, 0:1], 1.0)
    node_mean = xs_ref[...] / cnt
    edge_mean = ae / cnt
    feat = jnp.concatenate([u_ref[...], node_mean, edge_mean], axis=1)
    h = _elu(_dot_nt(feat, win_ref[...]) + bin_ref[...])
    skip = h
    for l in range(NHID):
        h = _elu(_dot_nt(h, wh_ref[l]) + bh_ref[l])
    h = h + skip
    z = _dot_nt(h, wo_ref[...])
    mu = jnp.mean(z, axis=-1, keepdims=True)
    var = jnp.mean((z - mu) ** 2, axis=-1, keepdims=True)
    z = (z - mu) / jnp.sqrt(var + 1e-5) * g_ref[...] + beta_ref[...]
    out_ref[...] = _elu(z)


def _tc2(batch8, u, val, cnt, xs, cs, W_in, bin2, W_hid, bh3, W_out,
         g2, beta2):
    return pl.pallas_call(
        _tc2_body,
        out_shape=jax.ShapeDtypeStruct((B, OUT), jnp.float32),
    )(batch8, u, val, cnt, xs, cs, W_in, bin2, W_hid, bh3, W_out, g2, beta2)


def kernel(x, edge_index, edge_attr, u, batch, W_in, b_in, W_hid, b_hid,
           W_out, gamma, beta):
    ei3d = edge_index.reshape(2, NCHUNK, CH)
    val, cnt = _sc_edge_scatter(ei3d, edge_attr)
    val = val.reshape(NC, NRV, 128)
    cnt = cnt.reshape(NC, NRV, 128)

    batch3d = batch.reshape(NBLK, 1, CHN)
    batch8 = jnp.pad(batch, (0, NROW - N), constant_values=B)
    batch8 = batch8.reshape(NRV, NPK).T
    xs, cs = _tc1(x, batch3d)
    return _tc2(batch8, u, val, cnt, xs, cs, W_in, b_in[None, :], W_hid,
                b_hid[:, None, :], W_out, gamma[None, :], beta[None, :])


# speedup vs baseline: 7.6259x; 1.0303x over previous
import functools

import jax
import jax.numpy as jnp
from jax import lax
from jax.experimental import pallas as pl
from jax.experimental.pallas import tpu as pltpu
from jax.experimental.pallas import tpu_sc as plsc

N = 10000
E = 160000
B = 64
DF = 256
DE = 16
GIN = 64
HID = 512
OUT = 256
NHID = 3

NC = 2
NS = 16
NW = NC * NS
CH = 125
NCHUNK = E // CH
TILE_C = NCHUNK // NW
EPT = TILE_C * CH
NROW = 10240
RPT = NROW // NS


def _sc_edge_scatter(ei3d, attr):
    mesh = plsc.VectorSubcoreMesh(core_axis_name="c", subcore_axis_name="s")

    @functools.partial(
        pl.kernel,
        out_type=(
            jax.ShapeDtypeStruct((NC, NROW, DE), jnp.float32),
            jax.ShapeDtypeStruct((NC, NROW, DE), jnp.float32),
        ),
        mesh=mesh,
        compiler_params=pltpu.CompilerParams(use_tc_tiling_on_sc=False),
        scratch_types=[
            pltpu.VMEM((TILE_C, CH), jnp.int32),
            pltpu.VMEM((EPT, DE), jnp.float32),
            pltpu.VMEM((RPT, DE), jnp.float32),
            pltpu.VMEM((CH, DE), jnp.float32),
            pltpu.VMEM_SHARED((NROW, DE), jnp.float32),
            pltpu.VMEM_SHARED((NROW, DE), jnp.float32),
            pltpu.SemaphoreType.DMA,
            pltpu.SemaphoreType.DMA,
        ],
    )
    def k(dst_hbm, attr_hbm, val_out, cnt_out,
          idx_all, attr_all, zrow, ones_rows,
          val_sh, cnt_sh, sem_v, sem_c):
        c = lax.axis_index("c")
        s = lax.axis_index("s")
        wid = c * NS + s

        zeros16 = jnp.zeros((16,), jnp.float32)
        lane = lax.iota(jnp.int32, 16)
        onehot = jnp.where(lane == 0, 1.0, 0.0).astype(jnp.float32)

        def _zero_zrow(i, carry):
            zrow[i, :] = zeros16
            return carry
        lax.fori_loop(0, RPT, _zero_zrow, 0)

        def _fill_ones(i, carry):
            ones_rows[i, :] = onehot
            return carry
        lax.fori_loop(0, CH, _fill_ones, 0)

        pltpu.sync_copy(zrow, val_sh.at[pl.ds(s * RPT, RPT)])
        pltpu.sync_copy(zrow, cnt_sh.at[pl.ds(s * RPT, RPT)])

        plsc.subcore_barrier()

        pltpu.sync_copy(dst_hbm.at[1, pl.ds(wid * TILE_C, TILE_C)], idx_all)
        pltpu.sync_copy(attr_hbm.at[pl.ds(wid * EPT, EPT)], attr_all)

        WAVE = 8

        def _wave(w, carry):
            hs = []
            for j in range(WAVE):
                ci = w * WAVE + j
                hs.append(pltpu.async_copy(
                    attr_all.at[pl.ds(ci * CH, CH)],
                    val_sh.at[idx_all.at[ci]], sem_v, add=True))
                hs.append(pltpu.async_copy(
                    ones_rows, cnt_sh.at[idx_all.at[ci]], sem_c, add=True))
            for h in hs:
                h.wait()
            return carry
        lax.fori_loop(0, TILE_C // WAVE, _wave, 0)

        plsc.subcore_barrier()

        pltpu.sync_copy(val_sh.at[pl.ds(s * RPT, RPT)],
                        val_out.at[c, pl.ds(s * RPT, RPT)])
        pltpu.sync_copy(cnt_sh.at[pl.ds(s * RPT, RPT)],
                        cnt_out.at[c, pl.ds(s * RPT, RPT)])

    return k(ei3d, attr)


NBLK = 5
CHN = N // NBLK
NRV = NROW * DE // 128
NPK = 128 // DE


def _elu(v):
    return jnp.where(v > 0, v, jnp.exp(v) - 1.0)


def _dot_nt(a, w):
    return lax.dot_general(a, w, (((1,), (1,)), ((), ())),
                           preferred_element_type=jnp.float32)


def _tc1_body(x_ref, b_ref, xs_ref, cs_ref, accx, accc):
    i = pl.program_id(0)

    @pl.when(i == 0)
    def _():
        accx[...] = jnp.zeros_like(accx)
        accc[...] = jnp.zeros_like(accc)

    bblk = b_ref[0]
    iota = lax.broadcasted_iota(jnp.int32, (B, CHN), 0)
    mask = (bblk == iota).astype(jnp.float32)
    accx[...] += jnp.dot(mask, x_ref[...], preferred_element_type=jnp.float32)
    accc[...] += jnp.sum(mask, axis=1, keepdims=True)

    @pl.when(i == NBLK - 1)
    def _():
        xs_ref[...] = accx[...]
        cs_ref[...] = jnp.broadcast_to(accc[...], (B, 128))


def _tc1(x, batch3d):
    return pl.pallas_call(
        _tc1_body,
        grid=(NBLK,),
        in_specs=[
            pl.BlockSpec((CHN, DF), lambda i: (i, 0)),
            pl.BlockSpec((1, 1, CHN), lambda i: (i, 0, 0)),
        ],
        out_specs=[pl.BlockSpec((B, DF), lambda i: (0, 0)),
                   pl.BlockSpec((B, 128), lambda i: (0, 0))],
        out_shape=[jax.ShapeDtypeStruct((B, DF), jnp.float32),
                   jax.ShapeDtypeStruct((B, 128), jnp.float32)],
        scratch_shapes=[
            pltpu.VMEM((B, DF), jnp.float32),
            pltpu.VMEM((B, 1), jnp.float32),
        ],
    )(x, batch3d)


def _tc2_body(b8_ref, u_ref, val_ref, cnt_ref, xs_ref, cs_ref, win_ref,
              bin_ref, wh_ref, bh_ref, wo_ref, g_ref, beta_ref, out_ref):
    c128 = cnt_ref[0] + cnt_ref[1]
    kk = lax.broadcasted_iota(jnp.int32, (128, 128), 0)
    ll = lax.broadcasted_iota(jnp.int32, (128, 128), 1)
    sel = (kk == (ll // DE) * DE).astype(jnp.float32)
    cb = jnp.dot(c128, sel, preferred_element_type=jnp.float32)
    em128 = (val_ref[0] + val_ref[1]) / jnp.maximum(cb, 1.0)
    iota8 = lax.broadcasted_iota(jnp.int32, (B, NRV), 0)
    ae = jnp.zeros((B, DE), jnp.float32)
    for a in range(NPK):
        mask_a = (b8_ref[a:a + 1, :] == iota8).astype(jnp.float32)
        ae = ae + jnp.dot(mask_a, em128[:, a * DE:(a + 1) * DE],
                          preferred_element_type=jnp.float32)

    cnt = jnp.maximum(cs_ref[:, 0:1], 1.0)
    node_mean = xs_ref[...] / cnt
    edge_mean = ae / cnt
    feat = jnp.concatenate([u_ref[...], node_mean, edge_mean], axis=1)
    h = _elu(_dot_nt(feat, win_ref[...]) + bin_ref[...])
    skip = h
    for l in range(NHID):
        h = _elu(_dot_nt(h, wh_ref[l]) + bh_ref[l])
    h = h + skip
    z = _dot_nt(h, wo_ref[...])
    mu = jnp.mean(z, axis=-1, keepdims=True)
    var = jnp.mean((z - mu) ** 2, axis=-1, keepdims=True)
    z = (z - mu) / jnp.sqrt(var + 1e-5) * g_ref[...] + beta_ref[...]
    out_ref[...] = _elu(z)


def _tc2(batch8, u, val, cnt, xs, cs, W_in, bin2, W_hid, bh3, W_out,
         g2, beta2):
    return pl.pallas_call(
        _tc2_body,
        out_shape=jax.ShapeDtypeStruct((B, OUT), jnp.float32),
    )(batch8, u, val, cnt, xs, cs, W_in, bin2, W_hid, bh3, W_out, g2, beta2)


def kernel(x, edge_index, edge_attr, u, batch, W_in, b_in, W_hid, b_hid,
           W_out, gamma, beta):
    ei3d = edge_index.reshape(2, NCHUNK, CH)
    val, cnt = _sc_edge_scatter(ei3d, edge_attr)
    val = val.reshape(NC, NRV, 128)
    cnt = cnt.reshape(NC, NRV, 128)

    batch3d = batch.reshape(NBLK, 1, CHN)
    batch8 = jnp.pad(batch, (0, NROW - N), constant_values=B)
    batch8 = batch8.reshape(NRV, NPK).T
    xs, cs = _tc1(x, batch3d)
    return _tc2(batch8, u, val, cnt, xs, cs, W_in, b_in[None, :], W_hid,
                b_hid[:, None, :], W_out, gamma[None, :], beta[None, :])


# speedup vs baseline: 8.0874x; 1.0605x over previous
import functools

import jax
import jax.numpy as jnp
from jax import lax
from jax.experimental import pallas as pl
from jax.experimental.pallas import tpu as pltpu
from jax.experimental.pallas import tpu_sc as plsc

N = 10000
E = 160000
B = 64
DF = 256
DE = 16
GIN = 64
HID = 512
OUT = 256
NHID = 3

NC = 2
NS = 16
NW = NC * NS
CH = 125
NCHUNK = E // CH
TILE_C = NCHUNK // NW
EPT = TILE_C * CH
NROW = 10240
RPT = NROW // NS


def _sc_edge_scatter(ei3d, attr):
    mesh = plsc.VectorSubcoreMesh(core_axis_name="c", subcore_axis_name="s")

    @functools.partial(
        pl.kernel,
        out_type=(
            jax.ShapeDtypeStruct((NC, NROW, DE), jnp.float32),
            jax.ShapeDtypeStruct((NC, NROW, DE), jnp.float32),
        ),
        mesh=mesh,
        compiler_params=pltpu.CompilerParams(use_tc_tiling_on_sc=False),
        scratch_types=[
            pltpu.VMEM((TILE_C, CH), jnp.int32),
            pltpu.VMEM((EPT, DE), jnp.float32),
            pltpu.VMEM((RPT, DE), jnp.float32),
            pltpu.VMEM((CH, DE), jnp.float32),
            pltpu.VMEM_SHARED((NROW, DE), jnp.float32),
            pltpu.VMEM_SHARED((NROW, DE), jnp.float32),
            pltpu.SemaphoreType.DMA,
            pltpu.SemaphoreType.DMA,
        ],
    )
    def k(dst_hbm, attr_hbm, val_out, cnt_out,
          idx_all, attr_all, zrow, ones_rows,
          val_sh, cnt_sh, sem_v, sem_c):
        c = lax.axis_index("c")
        s = lax.axis_index("s")
        wid = c * NS + s

        zeros16 = jnp.zeros((16,), jnp.float32)
        lane = lax.iota(jnp.int32, 16)
        onehot = jnp.where(lane == 0, 1.0, 0.0).astype(jnp.float32)

        h_idx = pltpu.async_copy(dst_hbm.at[1, pl.ds(wid * TILE_C, TILE_C)],
                                 idx_all, sem_v)
        h_attr = pltpu.async_copy(attr_hbm.at[pl.ds(wid * EPT, EPT)],
                                  attr_all, sem_c)

        def _zero_zrow(i, carry):
            zrow[i, :] = zeros16
            return carry
        lax.fori_loop(0, RPT, _zero_zrow, 0)

        def _fill_ones(i, carry):
            ones_rows[i, :] = onehot
            return carry
        lax.fori_loop(0, CH, _fill_ones, 0)

        pltpu.sync_copy(zrow, val_sh.at[pl.ds(s * RPT, RPT)])
        pltpu.sync_copy(zrow, cnt_sh.at[pl.ds(s * RPT, RPT)])

        plsc.subcore_barrier()
        h_idx.wait()
        h_attr.wait()

        WAVE = 20

        def _wave(w, carry):
            hs = []
            for j in range(WAVE):
                ci = w * WAVE + j
                hs.append(pltpu.async_copy(
                    attr_all.at[pl.ds(ci * CH, CH)],
                    val_sh.at[idx_all.at[ci]], sem_v, add=True))
                hs.append(pltpu.async_copy(
                    ones_rows, cnt_sh.at[idx_all.at[ci]], sem_c, add=True))
            for h in hs:
                h.wait()
            return carry
        lax.fori_loop(0, TILE_C // WAVE, _wave, 0)

        plsc.subcore_barrier()

        pltpu.sync_copy(val_sh.at[pl.ds(s * RPT, RPT)],
                        val_out.at[c, pl.ds(s * RPT, RPT)])
        pltpu.sync_copy(cnt_sh.at[pl.ds(s * RPT, RPT)],
                        cnt_out.at[c, pl.ds(s * RPT, RPT)])

    return k(ei3d, attr)


NBLK = 5
CHN = N // NBLK
NRV = NROW * DE // 128
NPK = 128 // DE


def _elu(v):
    return jnp.where(v > 0, v, jnp.exp(v) - 1.0)


def _dot_nt(a, w):
    return lax.dot_general(a, w, (((1,), (1,)), ((), ())),
                           preferred_element_type=jnp.float32)


def _tc1_body(x_ref, b_ref, xs_ref, cs_ref, accx, accc):
    i = pl.program_id(0)

    @pl.when(i == 0)
    def _():
        accx[...] = jnp.zeros_like(accx)
        accc[...] = jnp.zeros_like(accc)

    bblk = b_ref[0]
    iota = lax.broadcasted_iota(jnp.int32, (B, CHN), 0)
    mask = (bblk == iota).astype(jnp.float32)
    accx[...] += jnp.dot(mask, x_ref[...], preferred_element_type=jnp.float32)
    accc[...] += jnp.sum(mask, axis=1, keepdims=True)

    @pl.when(i == NBLK - 1)
    def _():
        xs_ref[...] = accx[...]
        cs_ref[...] = jnp.broadcast_to(accc[...], (B, 128))


def _tc1(x, batch3d):
    return pl.pallas_call(
        _tc1_body,
        grid=(NBLK,),
        in_specs=[
            pl.BlockSpec((CHN, DF), lambda i: (i, 0)),
            pl.BlockSpec((1, 1, CHN), lambda i: (i, 0, 0)),
        ],
        out_specs=[pl.BlockSpec((B, DF), lambda i: (0, 0)),
                   pl.BlockSpec((B, 128), lambda i: (0, 0))],
        out_shape=[jax.ShapeDtypeStruct((B, DF), jnp.float32),
                   jax.ShapeDtypeStruct((B, 128), jnp.float32)],
        scratch_shapes=[
            pltpu.VMEM((B, DF), jnp.float32),
            pltpu.VMEM((B, 1), jnp.float32),
        ],
    )(x, batch3d)


def _tc2_body(b8_ref, u_ref, val_ref, cnt_ref, xs_ref, cs_ref, win_ref,
              bin_ref, wh_ref, bh_ref, wo_ref, g_ref, beta_ref, out_ref):
    c128 = cnt_ref[0] + cnt_ref[1]
    kk = lax.broadcasted_iota(jnp.int32, (128, 128), 0)
    ll = lax.broadcasted_iota(jnp.int32, (128, 128), 1)
    sel = (kk == (ll // DE) * DE).astype(jnp.float32)
    cb = jnp.dot(c128, sel, preferred_element_type=jnp.float32)
    em128 = (val_ref[0] + val_ref[1]) / jnp.maximum(cb, 1.0)
    iota8 = lax.broadcasted_iota(jnp.int32, (B, NRV), 0)
    ae = jnp.zeros((B, DE), jnp.float32)
    for a in range(NPK):
        mask_a = (b8_ref[a:a + 1, :] == iota8).astype(jnp.float32)
        ae = ae + jnp.dot(mask_a, em128[:, a * DE:(a + 1) * DE],
                          preferred_element_type=jnp.float32)

    cnt = jnp.maximum(cs_ref[:, 0:1], 1.0)
    node_mean = xs_ref[...] / cnt
    edge_mean = ae / cnt
    feat = jnp.concatenate([u_ref[...], node_mean, edge_mean], axis=1)
    h = _elu(_dot_nt(feat, win_ref[...]) + bin_ref[...])
    skip = h
    for l in range(NHID):
        h = _elu(_dot_nt(h, wh_ref[l]) + bh_ref[l])
    h = h + skip
    z = _dot_nt(h, wo_ref[...])
    mu = jnp.mean(z, axis=-1, keepdims=True)
    var = jnp.mean((z - mu) ** 2, axis=-1, keepdims=True)
    z = (z - mu) / jnp.sqrt(var + 1e-5) * g_ref[...] + beta_ref[...]
    out_ref[...] = _elu(z)


def _tc2(batch8, u, val, cnt, xs, cs, W_in, bin2, W_hid, bh3, W_out,
         g2, beta2):
    return pl.pallas_call(
        _tc2_body,
        out_shape=jax.ShapeDtypeStruct((B, OUT), jnp.float32),
    )(batch8, u, val, cnt, xs, cs, W_in, bin2, W_hid, bh3, W_out, g2, beta2)


def kernel(x, edge_index, edge_attr, u, batch, W_in, b_in, W_hid, b_hid,
           W_out, gamma, beta):
    ei3d = edge_index.reshape(2, NCHUNK, CH)
    val, cnt = _sc_edge_scatter(ei3d, edge_attr)
    val = val.reshape(NC, NRV, 128)
    cnt = cnt.reshape(NC, NRV, 128)

    batch3d = batch.reshape(NBLK, 1, CHN)
    batch8 = jnp.pad(batch, (0, NROW - N), constant_values=B)
    batch8 = batch8.reshape(NRV, NPK).T
    xs, cs = _tc1(x, batch3d)
    return _tc2(batch8, u, val, cnt, xs, cs, W_in, b_in[None, :], W_hid,
                b_hid[:, None, :], W_out, gamma[None, :], beta[None, :])
